# Initial kernel scaffold; baseline (speedup 1.0000x reference)
import jax
import jax.numpy as jnp
from jax.experimental import pallas as pl


def kernel(X, user_id_mapping, user_table, item_table, W, b):
    raise NotImplementedError("write your pallas kernel here")


# speedup vs baseline: 3.2675x; 3.2675x over previous
import functools

import jax
import jax.numpy as jnp
from jax import lax
from jax.experimental import pallas as pl
from jax.experimental.pallas import tpu as pltpu
from jax.experimental.pallas import tpu_sc as plsc

NC = 2
NS = 16
L = 16
NW = NC * NS
B = 16384
D = 128
BPW = B // NW
CH = 128
NCHUNK = BPW // CH


def _sc_body(x_hbm, map_hbm, ut_hbm, it_hbm, w_hbm, b_hbm, out_hbm,
             xv, map_v, uidx, iidx, wv, bv, outv,
             ubuf0, ubuf1, ibuf0, ibuf1,
             sem_u0, sem_u1, sem_i0, sem_i1):
  cid = lax.axis_index("c")
  sid = lax.axis_index("s")
  wid = sid * NC + cid

  pltpu.sync_copy(x_hbm.at[wid], xv)
  pltpu.sync_copy(map_hbm, map_v)
  pltpu.sync_copy(w_hbm, wv)
  pltpu.sync_copy(b_hbm, bv)

  lanes = jnp.arange(L, dtype=jnp.int32)
  for g in range(BPW // L):
    col = lanes * 2 + (g * 2 * L)
    raw_u = plsc.load_gather(xv, [col])
    raw_i = plsc.load_gather(xv, [col + 1])
    mapped = plsc.load_gather(map_v, [raw_u])
    uidx[g // (CH // L), pl.ds((g % (CH // L)) * L, L)] = mapped
    iidx[g // (CH // L), pl.ds((g % (CH // L)) * L, L)] = raw_i

  ubufs = (ubuf0, ubuf1)
  ibufs = (ibuf0, ibuf1)
  usems = (sem_u0, sem_u1)
  isems = (sem_i0, sem_i1)

  def fire(j):
    s = j % 2
    cu = pltpu.async_copy(ut_hbm.at[uidx.at[j]], ubufs[s], usems[s])
    ci = pltpu.async_copy(it_hbm.at[iidx.at[j]], ibufs[s], isems[s])
    return cu, ci

  wk = [wv[pl.ds(k * L, L)] for k in range(D // L)]
  acc0 = jnp.where(lanes == 0, bv[pl.ds(0, L)], jnp.zeros((L,), jnp.float32))

  pend = fire(0)
  for j in range(NCHUNK):
    cu, ci = pend
    if j + 1 < NCHUNK:
      pend = fire(j + 1)
    cu.wait()
    ci.wait()
    s = j % 2
    ub, ib = ubufs[s], ibufs[s]

    def grp_body(g, _):
      res = jnp.zeros((L,), jnp.float32)
      for r in range(L):
        row = g * L + r
        acc = acc0
        for k in range(D // L):
          u = ub[row, pl.ds(k * L, L)]
          v = ib[row, pl.ds(k * L, L)]
          acc = acc + (u * v) * wk[k]
        res = jnp.where(lanes == r, jnp.sum(acc), res)
      outv[pl.ds(j * CH + g * L, L)] = res
      return 0

    lax.fori_loop(0, CH // L, grp_body, 0)

  pltpu.sync_copy(outv, out_hbm.at[wid])


@jax.jit
def _run(x_r, user_id_mapping, user_table, item_table, w_r, b_pad):
  mesh = plsc.VectorSubcoreMesh(core_axis_name="c", subcore_axis_name="s")
  f = pl.kernel(
      _sc_body,
      out_type=jax.ShapeDtypeStruct((NW, BPW), jnp.float32),
      mesh=mesh,
      compiler_params=pltpu.CompilerParams(needs_layout_passes=False),
      scratch_types=[
          pltpu.VMEM((BPW * 2,), jnp.int32),
          pltpu.VMEM((256,), jnp.int32),
          pltpu.VMEM((NCHUNK, CH), jnp.int32),
          pltpu.VMEM((NCHUNK, CH), jnp.int32),
          pltpu.VMEM((D,), jnp.float32),
          pltpu.VMEM((16,), jnp.float32),
          pltpu.VMEM((BPW,), jnp.float32),
          pltpu.VMEM((CH, D), jnp.float32),
          pltpu.VMEM((CH, D), jnp.float32),
          pltpu.VMEM((CH, D), jnp.float32),
          pltpu.VMEM((CH, D), jnp.float32),
          pltpu.SemaphoreType.DMA,
          pltpu.SemaphoreType.DMA,
          pltpu.SemaphoreType.DMA,
          pltpu.SemaphoreType.DMA,
      ],
  )
  return f(x_r, user_id_mapping, user_table, item_table, w_r, b_pad)


def kernel(X, user_id_mapping, user_table, item_table, W, b):
  x_r = X.reshape(NW, BPW * 2)
  w_r = W.reshape(D)
  b_pad = jnp.pad(b.astype(jnp.float32), (0, 15))
  out = _run(x_r, user_id_mapping, user_table, item_table, w_r, b_pad)
  return out.reshape(B, 1)
